# baseline (speedup 1.0000x reference)
import functools

import jax
import jax.numpy as jnp
from jax import lax
from jax.experimental import pallas as pl
from jax.experimental.pallas import tpu as pltpu
from jax.experimental.pallas import tpu_sc as plsc

EMBED_DIM = 64
NC = 2
NS = 16
NW = NC * NS


def _gather_call(ntot: int, chunk: int, nch: int):
    mesh = plsc.VectorSubcoreMesh(core_axis_name="c", subcore_axis_name="s")
    per_w = ntot // NW

    @functools.partial(
        pl.kernel,
        mesh=mesh,
        out_type=jax.ShapeDtypeStruct((ntot, EMBED_DIM), jnp.float32),
        scratch_types=[
            pltpu.VMEM((nch, chunk), jnp.int32),
            pltpu.VMEM((chunk, EMBED_DIM), jnp.float32),
            pltpu.SemaphoreType.DMA,
        ],
        compiler_params=pltpu.CompilerParams(use_tc_tiling_on_sc=False),
    )
    def k(idx_hbm, table_hbm, out_hbm, idx_v, rows_v, sem):
        wid = lax.axis_index("s") * NC + lax.axis_index("c")
        base = wid * per_w
        pltpu.sync_copy(idx_hbm.at[wid], idx_v)

        def body(j, _):
            pltpu.async_copy(table_hbm.at[idx_v.at[j]], rows_v, sem).wait()
            pltpu.sync_copy(rows_v, out_hbm.at[pl.ds(base + j * chunk, chunk)])
            return 0

        lax.fori_loop(0, nch, body, 0)

    return k


def kernel(nodes, table):
    batch, fields = nodes.shape
    ntot = batch * fields
    chunk = 128
    nch = ntot // NW // chunk
    idx3 = nodes.reshape(NW, nch, chunk)
    out = _gather_call(ntot, chunk, nch)(idx3, table)
    return out.reshape(batch, fields, EMBED_DIM)


# speedup vs baseline: 1.0763x; 1.0763x over previous
import functools

import jax
import jax.numpy as jnp
from jax import lax
from jax.experimental import pallas as pl
from jax.experimental.pallas import tpu as pltpu
from jax.experimental.pallas import tpu_sc as plsc

EMBED_DIM = 64
NC = 2
NS = 16
NW = NC * NS
CHUNK = 128
NBUF = 8


def _gather_call(ntot: int, nch: int):
    mesh = plsc.VectorSubcoreMesh(core_axis_name="c", subcore_axis_name="s")
    per_w = ntot // NW
    nouter = nch // NBUF

    @functools.partial(
        pl.kernel,
        mesh=mesh,
        out_type=jax.ShapeDtypeStruct((ntot, EMBED_DIM), jnp.float32),
        scratch_types=[
            pltpu.VMEM((nch, CHUNK), jnp.int32),
            [pltpu.VMEM((CHUNK, EMBED_DIM), jnp.float32) for _ in range(NBUF)],
            [pltpu.SemaphoreType.DMA for _ in range(NBUF)],
            [pltpu.SemaphoreType.DMA for _ in range(NBUF)],
        ],
        compiler_params=pltpu.CompilerParams(use_tc_tiling_on_sc=False),
    )
    def k(idx_hbm, table_hbm, out_hbm, idx_v, rows, gsem, wsem):
        wid = lax.axis_index("s") * NC + lax.axis_index("c")
        base = wid * per_w
        pltpu.sync_copy(idx_hbm.at[wid], idx_v)

        def gather(b, j):
            return pltpu.make_async_copy(
                table_hbm.at[idx_v.at[j]], rows[b], gsem[b])

        def writeback(b, j):
            return pltpu.make_async_copy(
                rows[b], out_hbm.at[pl.ds(base + j * CHUNK, CHUNK)], wsem[b])

        for b in range(NBUF):
            gather(b, b).start()

        def outer(jo, _):
            for b in range(NBUF):
                j = jo * NBUF + b
                gather(b, j).wait()
                writeback(b, j).start()
                writeback(b, j).wait()
                gather(b, j + NBUF).start()
            return 0

        lax.fori_loop(0, nouter - 1, outer, 0)

        for b in range(NBUF):
            j = (nouter - 1) * NBUF + b
            gather(b, j).wait()
            writeback(b, j).start()
        for b in range(NBUF):
            j = (nouter - 1) * NBUF + b
            writeback(b, j).wait()

    return k


def kernel(nodes, table):
    batch, fields = nodes.shape
    ntot = batch * fields
    nch = ntot // NW // CHUNK
    idx3 = nodes.reshape(NW, nch, CHUNK)
    out = _gather_call(ntot, nch)(idx3, table)
    return out.reshape(batch, fields, EMBED_DIM)
